# Initial kernel scaffold; baseline (speedup 1.0000x reference)
import jax
import jax.numpy as jnp
from jax.experimental import pallas as pl


def kernel(x, edge_index, W1, b1, W2, b2, Wd1, bd1, Wd2, bd2):
    raise NotImplementedError("write your pallas kernel here")


# speedup vs baseline: 18.9712x; 18.9712x over previous
import functools

import jax
import jax.numpy as jnp
from jax import lax
from jax.experimental import pallas as pl
from jax.experimental.pallas import tpu as pltpu
import jax.experimental.pallas.tpu_sc as plsc

N = 10000
E = 640000
D_IN, D_HID, D_LAT = 128, 64, 32

NC, NS = 2, 16
NW = NC * NS
CW = 128
EP = 655360
RW = EP // CW // NW
NP = 10240
RB = NP // NS


def _sc_mesh():
    return plsc.VectorSubcoreMesh(core_axis_name="c", subcore_axis_name="s")


_SC_PARAMS = pltpu.CompilerParams(use_tc_tiling_on_sc=False)


@functools.partial(
    pl.kernel,
    out_type=jax.ShapeDtypeStruct((NC, NP, 1), jnp.float32),
    mesh=_sc_mesh(),
    compiler_params=_SC_PARAMS,
    scratch_types=[
        pltpu.VMEM((RW, CW), jnp.int32),
        pltpu.VMEM((CW, 1), jnp.float32),
        pltpu.VMEM_SHARED((NP, 1), jnp.float32),
    ],
)
def _deg_kernel(dstp, ones, zer, out, dstv, ones_v, deg_s):
    cid = lax.axis_index("c")
    sid = lax.axis_index("s")
    wid = sid * NC + cid
    r0 = sid * RB
    pltpu.sync_copy(zer.at[pl.ds(r0, RB)], deg_s.at[pl.ds(r0, RB)])
    pltpu.sync_copy(ones, ones_v)
    pltpu.sync_copy(dstp.at[pl.ds(wid * RW, RW)], dstv)
    plsc.subcore_barrier()

    def body(j, carry):
        pltpu.sync_copy(ones_v, deg_s.at[dstv.at[j]], add=True)
        return carry

    lax.fori_loop(0, RW, body, 0)
    plsc.subcore_barrier()
    pltpu.sync_copy(deg_s.at[pl.ds(r0, RB)], out.at[cid, pl.ds(r0, RB)])


def _make_agg_kernel(D):
    @functools.partial(
        pl.kernel,
        out_type=jax.ShapeDtypeStruct((NC, NP, D), jnp.float32),
        mesh=_sc_mesh(),
        compiler_params=_SC_PARAMS,
        scratch_types=[
            pltpu.VMEM((RW, CW), jnp.int32),
            pltpu.VMEM((RW, CW), jnp.int32),
            pltpu.VMEM((CW, D), jnp.float32),
            pltpu.VMEM_SHARED((NP, D), jnp.float32),
            pltpu.SemaphoreType.DMA,
        ],
    )
    def agg(hs, srcp, dstp, zer, out, srcv, dstv, gbuf, agg_s, sem):
        cid = lax.axis_index("c")
        sid = lax.axis_index("s")
        wid = sid * NC + cid
        r0 = sid * RB
        pltpu.sync_copy(zer.at[pl.ds(r0, RB)], agg_s.at[pl.ds(r0, RB)])
        pltpu.sync_copy(srcp.at[pl.ds(wid * RW, RW)], srcv)
        pltpu.sync_copy(dstp.at[pl.ds(wid * RW, RW)], dstv)
        plsc.subcore_barrier()

        def body(j, carry):
            pltpu.async_copy(hs.at[srcv.at[j]], gbuf, sem).wait()
            pltpu.sync_copy(gbuf, agg_s.at[dstv.at[j]], add=True)
            return carry

        lax.fori_loop(0, RW, body, 0)
        plsc.subcore_barrier()
        pltpu.sync_copy(agg_s.at[pl.ds(r0, RB)], out.at[cid, pl.ds(r0, RB)])

    return agg


_agg64 = _make_agg_kernel(D_HID)
_agg32 = _make_agg_kernel(D_LAT)


def _tc1_body(x_ref, w_ref, degp_ref, hs_ref, dis_ref):
    dp = degp_ref[...]
    deg = dp[0, :N, :] + dp[1, :N, :] + 1.0
    dis = lax.rsqrt(deg)
    h = jnp.dot(x_ref[...], w_ref[...], preferred_element_type=jnp.float32)
    hs_ref[...] = h * dis
    dis_ref[...] = dis


def _tc1(x, W1, degp):
    return pl.pallas_call(
        _tc1_body,
        out_shape=(
            jax.ShapeDtypeStruct((N, D_HID), jnp.float32),
            jax.ShapeDtypeStruct((N, 1), jnp.float32),
        ),
    )(x, W1, degp)


def _tc2_body(aggp_ref, hs_ref, dis_ref, b_ref, w_ref, out_ref):
    ap = aggp_ref[...]
    dis = dis_ref[...]
    a = ap[0, :N, :] + ap[1, :N, :] + hs_ref[...]
    h = jnp.maximum(a * dis + b_ref[...][None, :], 0.0)
    h2 = jnp.dot(h, w_ref[...], preferred_element_type=jnp.float32)
    out_ref[...] = h2 * dis


def _tc2(agg1, hs1, dis, b1, W2):
    return pl.pallas_call(
        _tc2_body,
        out_shape=jax.ShapeDtypeStruct((N, D_LAT), jnp.float32),
    )(agg1, hs1, dis, b1, W2)


def _tc3_body(aggp_ref, hs_ref, dis_ref, b2_ref, wd1_ref, bd1_ref,
              wd2_ref, bd2_ref, out_ref):
    ap = aggp_ref[...]
    dis = dis_ref[...]
    z = jnp.maximum((ap[0, :N, :] + ap[1, :N, :] + hs_ref[...]) * dis
                    + b2_ref[...][None, :], 0.0)
    d = jnp.maximum(
        jnp.dot(z, wd1_ref[...], preferred_element_type=jnp.float32)
        + bd1_ref[...][None, :], 0.0)
    t = (jnp.dot(d, wd2_ref[...], preferred_element_type=jnp.float32)
         + bd2_ref[...][None, :])
    out_ref[...] = 1.0 / (1.0 + jnp.exp(-t))


def _tc3(agg2, hs2, dis, b2, Wd1, bd1, Wd2, bd2):
    return pl.pallas_call(
        _tc3_body,
        out_shape=jax.ShapeDtypeStruct((N, D_IN), jnp.float32),
    )(agg2, hs2, dis, b2, Wd1, bd1, Wd2, bd2)


def kernel(x, edge_index, W1, b1, W2, b2, Wd1, bd1, Wd2, bd2):
    src = edge_index[0].astype(jnp.int32)
    dst = edge_index[1].astype(jnp.int32)
    pad = EP - E
    srcp = jnp.concatenate([src, jnp.zeros((pad,), jnp.int32)]).reshape(EP // CW, CW)
    dstp = jnp.concatenate([dst, jnp.full((pad,), N, jnp.int32)]).reshape(EP // CW, CW)
    ones = jnp.ones((CW, 1), jnp.float32)
    zer1 = jnp.zeros((NP, 1), jnp.float32)
    zer64 = jnp.zeros((NP, D_HID), jnp.float32)
    zer32 = jnp.zeros((NP, D_LAT), jnp.float32)

    degp = _deg_kernel(dstp, ones, zer1)
    hs1, dis = _tc1(x, W1, degp)
    agg1 = _agg64(hs1, srcp, dstp, zer64)
    hs2 = _tc2(agg1, hs1, dis, b1, W2)
    agg2 = _agg32(hs2, srcp, dstp, zer32)
    return _tc3(agg2, hs2, dis, b2, Wd1, bd1, Wd2, bd2)
